# baseline (speedup 1.0000x reference)
import jax
import jax.numpy as jnp
from jax.experimental import pallas as pl

N = 10000
K = 32
D = 128
OUT = 128

BN = 200


def _fused_body(mb_ref, nf_ref, w1_ref, w2_ref, b_ref, out_ref):
    acc = mb_ref[:, 0, :]
    for k in range(1, K):
        acc = jnp.maximum(acc, mb_ref[:, k, :])
    out = jnp.dot(acc, w1_ref[...], preferred_element_type=jnp.float32)
    out += jnp.dot(nf_ref[...], w2_ref[...], preferred_element_type=jnp.float32)
    out_ref[...] = out + b_ref[...]


def kernel(mailbox_h, node_feat, W, b):
    w1 = W[:, :D].T
    w2 = W[:, D:].T
    b2 = b.reshape(1, OUT)
    grid = N // BN
    return pl.pallas_call(
        _fused_body,
        grid=(grid,),
        in_specs=[
            pl.BlockSpec((BN, K, D), lambda i: (i, 0, 0)),
            pl.BlockSpec((BN, D), lambda i: (i, 0)),
            pl.BlockSpec((D, OUT), lambda i: (0, 0)),
            pl.BlockSpec((D, OUT), lambda i: (0, 0)),
            pl.BlockSpec((1, OUT), lambda i: (0, 0)),
        ],
        out_specs=pl.BlockSpec((BN, OUT), lambda i: (i, 0)),
        out_shape=jax.ShapeDtypeStruct((N, OUT), jnp.float32),
    )(mailbox_h, node_feat, w1, w2, b2)


# speedup vs baseline: 1.2880x; 1.2880x over previous
import jax
import jax.numpy as jnp
from jax.experimental import pallas as pl

N = 10000
K = 32
D = 128
OUT = 128

BN = 200


def _fused_body(mb_ref, nf_ref, w1_ref, w2_ref, b_ref, out_ref):
    v = mb_ref[...]
    w = jnp.maximum(
        jnp.maximum(v[:, 0:8, :], v[:, 8:16, :]),
        jnp.maximum(v[:, 16:24, :], v[:, 24:32, :]),
    )
    acc = jnp.max(w, axis=1)
    out = jnp.dot(acc, w1_ref[...], preferred_element_type=jnp.float32)
    out += jnp.dot(nf_ref[...], w2_ref[...], preferred_element_type=jnp.float32)
    out_ref[...] = out + b_ref[...]


def kernel(mailbox_h, node_feat, W, b):
    w1 = W[:, :D].T
    w2 = W[:, D:].T
    b2 = b.reshape(1, OUT)
    grid = N // BN
    return pl.pallas_call(
        _fused_body,
        grid=(grid,),
        in_specs=[
            pl.BlockSpec((BN, K, D), lambda i: (i, 0, 0)),
            pl.BlockSpec((BN, D), lambda i: (i, 0)),
            pl.BlockSpec((D, OUT), lambda i: (0, 0)),
            pl.BlockSpec((D, OUT), lambda i: (0, 0)),
            pl.BlockSpec((1, OUT), lambda i: (0, 0)),
        ],
        out_specs=pl.BlockSpec((BN, OUT), lambda i: (i, 0)),
        out_shape=jax.ShapeDtypeStruct((N, OUT), jnp.float32),
    )(mailbox_h, node_feat, w1, w2, b2)


# speedup vs baseline: 1.5805x; 1.2271x over previous
import jax
import jax.numpy as jnp
from jax.experimental import pallas as pl

N = 10000
K = 32
D = 128
OUT = 128

BN = 400


def _fused_body(mb_ref, nf_ref, w1_ref, w2_ref, b_ref, out_ref):
    v = mb_ref[...]
    w = jnp.maximum(
        jnp.maximum(v[:, 0:8, :], v[:, 8:16, :]),
        jnp.maximum(v[:, 16:24, :], v[:, 24:32, :]),
    )
    acc = jnp.max(w, axis=1)
    out = jnp.dot(acc, w1_ref[...], preferred_element_type=jnp.float32)
    out += jnp.dot(nf_ref[...], w2_ref[...], preferred_element_type=jnp.float32)
    out_ref[...] = out + b_ref[...]


def kernel(mailbox_h, node_feat, W, b):
    w1 = W[:, :D].T
    w2 = W[:, D:].T
    b2 = b.reshape(1, OUT)
    grid = N // BN
    return pl.pallas_call(
        _fused_body,
        grid=(grid,),
        in_specs=[
            pl.BlockSpec((BN, K, D), lambda i: (i, 0, 0)),
            pl.BlockSpec((BN, D), lambda i: (i, 0)),
            pl.BlockSpec((D, OUT), lambda i: (0, 0)),
            pl.BlockSpec((D, OUT), lambda i: (0, 0)),
            pl.BlockSpec((1, OUT), lambda i: (0, 0)),
        ],
        out_specs=pl.BlockSpec((BN, OUT), lambda i: (i, 0)),
        out_shape=jax.ShapeDtypeStruct((N, OUT), jnp.float32),
    )(mailbox_h, node_feat, w1, w2, b2)
